# baseline (speedup 1.0000x reference)
import functools

import jax
import jax.numpy as jnp
from jax.experimental import pallas as pl
from jax.experimental.pallas import tpu as pltpu
from jax.experimental.pallas import tpu_sc as plsc

_N = 10000
_D = 128
_C = 128
_N_LABELED = 1280
_K = int((_N - _N_LABELED) * 0.005)
_EPS = 1e-12
_INT_MIN = -(2 ** 31)
_B = 2000
_NB = _N // _B
_NSC = 400
_NTC = _N - _NSC
_BT = 1200
_NBT = _NTC // _BT
_NG = _NSC // 16
_NW = 32


def _rownorm(v):
    return v * jax.lax.rsqrt(jnp.maximum(jnp.sum(v * v, axis=1, keepdims=True), _EPS))


def _dot(a, b, ca, cb):
    return jax.lax.dot_general(
        a, b, (((ca,), (cb,)), ((), ())),
        preferred_element_type=jnp.float32,
        precision=jax.lax.Precision.HIGHEST,
    )


def _keys(probt):
    int_min = jnp.int32(_INT_MIN)
    i32 = jax.lax.bitcast_convert_type(probt, jnp.int32)
    return jnp.where(i32 >= 0, i32, jnp.bitwise_or(jnp.bitwise_not(i32), int_min))


def _kth_mask_t(probt):
    int_min = jnp.int32(_INT_MIN)
    s = _keys(probt)
    t = jnp.full((1, probt.shape[1]), _INT_MIN, jnp.int32)
    for b in range(31, -1, -1):
        inc = int_min if b == 31 else jnp.int32(1 << b)
        cand = t + inc
        cnt = jnp.sum((s >= cand).astype(jnp.int32), axis=0, keepdims=True)
        t = jnp.where(cnt >= _K, cand, t)
    return jnp.where(s >= t, probt, 0.0)


def _k1_body(x_ref, lab_ref, tot_ref, cnt_ref, pn_ref, probt_ref, skeys3_ref):
    i = pl.program_id(0)

    @pl.when(i == 0)
    def _():
        tot_ref[...] = jnp.zeros_like(tot_ref)
        cnt_ref[...] = jnp.zeros_like(cnt_ref)

    x = x_ref[...]
    tot_ref[...] += _dot(lab_ref[...], x, 0, 0)
    cnt_ref[...] += _dot(lab_ref[...], jnp.ones((_B, 1), jnp.float32), 0, 0)

    @pl.when(i == _NB - 1)
    def _():
        protos = tot_ref[...] * (1.0 / cnt_ref[...])
        pn = _rownorm(protos)
        pn_ref[...] = pn
        xn_tail = _rownorm(x[_B - _NSC:])
        probt = _dot(pn, xn_tail, 1, 1)
        probt_ref[...] = probt
        skeys = _keys(probt)
        for g in range(_NG):
            skeys3_ref[g] = skeys[:, 16 * g:16 * (g + 1)]


def _topk_tc_body(x_ref, um_ref, pn_ref, supx_ref, colup_ref):
    i = pl.program_id(0)

    @pl.when(i == 0)
    def _():
        supx_ref[...] = jnp.zeros_like(supx_ref)
        colup_ref[...] = jnp.zeros_like(colup_ref)

    x = x_ref[...]
    xn = _rownorm(x)
    probt = _dot(pn_ref[...], xn, 1, 1)
    upt = _kth_mask_t(probt) * um_ref[0]
    supx_ref[...] += _dot(upt, x, 1, 0)
    colup_ref[...] += _dot(upt, jnp.ones((_BT, 1), jnp.float32), 1, 0)


def _k3_body(x_ref, xsc_ref, um_sc_ref, probt_ref, skeys3_ref, t3_ref,
             tot_ref, cnt_ref, supx_ref, colup_ref, out_ref, pn2_ref):
    i = pl.program_id(0)

    @pl.when(i == 0)
    def _():
        probt = probt_ref[...]
        pieces = []
        for g in range(_NG):
            t_bc = jnp.broadcast_to(t3_ref[g:g + 1, :], (_C, 16))
            pieces.append(jnp.where(skeys3_ref[g] >= t_bc,
                                    probt[:, 16 * g:16 * (g + 1)], 0.0))
        upt = jnp.concatenate(pieces, axis=1) * um_sc_ref[0]
        supx = supx_ref[...] + _dot(upt, xsc_ref[...], 1, 0)
        colup = colup_ref[...] + _dot(upt, jnp.ones((_NSC, 1), jnp.float32), 1, 0)
        countc = cnt_ref[...]
        protos = tot_ref[...] * (1.0 / countc)
        denom = colup + countc
        protos2 = protos + (supx - protos * colup) / denom
        pn2_ref[...] = _rownorm(protos2)

    out_ref[...] = _dot(_rownorm(x_ref[...]), pn2_ref[...], 1, 1)


def _sc_call(skeys3):
    mesh = plsc.VectorSubcoreMesh(core_axis_name="c", subcore_axis_name="s")

    @functools.partial(
        pl.kernel,
        out_type=jax.ShapeDtypeStruct((_NG, 16), jnp.int32),
        mesh=mesh,
        scratch_types=[
            pltpu.VMEM((_C, 16), jnp.int32),
            pltpu.VMEM((16,), jnp.int32),
            pltpu.VMEM((16,), jnp.int32),
        ],
    )
    def sc_thresh(sk_hbm, t_hbm, buf, tbuf, cnt_ref):
        wid = jax.lax.axis_index("s") * 2 + jax.lax.axis_index("c")
        one = jnp.full((16,), 1, jnp.int32)
        zero = jnp.full((16,), 0, jnp.int32)

        @pl.loop(0, (_NG + _NW - 1) // _NW)
        def _(j):
            g = wid + j * _NW

            @pl.when(g < _NG)
            def _():
                pltpu.sync_copy(sk_hbm.at[g], buf)
                t = jnp.full((16,), _INT_MIN, jnp.int32)
                for b in range(31, -1, -1):
                    inc = jnp.int32(_INT_MIN) if b == 31 else jnp.int32(1 << b)
                    cand = t + inc
                    cnt_ref[...] = zero

                    @pl.loop(0, _C // 8)
                    def _(cc):
                        acc = zero
                        for u in range(8):
                            acc = acc + jnp.where(buf[cc * 8 + u] >= cand, one, zero)
                        cnt_ref[...] += acc

                    t = jnp.where(cnt_ref[...] >= _K, cand, t)
                tbuf[...] = t
                pltpu.sync_copy(tbuf, t_hbm.at[g])

    return sc_thresh(skeys3)


def _const_spec(r):
    return pl.BlockSpec((_C, r), lambda i: (0, 0))


def kernel(inputs, labels, labels_mask, unlabels_mask):
    del labels_mask
    f32 = jnp.float32
    um_tc = unlabels_mask[:_NTC].astype(f32).reshape(_NBT, 1, _BT)
    um_sc = unlabels_mask[_NTC:].astype(f32).reshape(1, 1, _NSC)
    x = inputs
    lab = labels.astype(f32)

    tot, cnt, pn, probt_sc, skeys3 = pl.pallas_call(
        _k1_body,
        grid=(_NB,),
        in_specs=[pl.BlockSpec((_B, _D), lambda i: (i, 0)),
                  pl.BlockSpec((_B, _D), lambda i: (i, 0))],
        out_specs=[_const_spec(_D), _const_spec(1), _const_spec(_D),
                   _const_spec(_NSC),
                   pl.BlockSpec((_NG, _C, 16), lambda i: (0, 0, 0))],
        out_shape=[jax.ShapeDtypeStruct((_C, _D), f32),
                   jax.ShapeDtypeStruct((_C, 1), f32),
                   jax.ShapeDtypeStruct((_C, _D), f32),
                   jax.ShapeDtypeStruct((_C, _NSC), f32),
                   jax.ShapeDtypeStruct((_NG, _C, 16), jnp.int32)],
    )(x, lab)

    def _tc_thresh_body(sk_ref, t_ref):
        for g in range(_NG):
            s = sk_ref[g]
            t = jnp.full((1, 16), _INT_MIN, jnp.int32)
            for b in range(31, -1, -1):
                inc = jnp.int32(_INT_MIN) if b == 31 else jnp.int32(1 << b)
                cand = t + inc
                cnt = jnp.sum((s >= cand).astype(jnp.int32), axis=0, keepdims=True)
                t = jnp.where(cnt >= _K, cand, t)
            t_ref[g] = t[0]

    t3 = pl.pallas_call(
        _tc_thresh_body,
        grid=(1,),
        in_specs=[pl.BlockSpec((_NG, _C, 16), lambda i: (0, 0, 0))],
        out_specs=pl.BlockSpec((_NG, 16), lambda i: (0, 0)),
        out_shape=jax.ShapeDtypeStruct((_NG, 16), jnp.int32),
    )(skeys3)

    supx, colup = pl.pallas_call(
        _topk_tc_body,
        grid=(_NBT,),
        in_specs=[pl.BlockSpec((_BT, _D), lambda i: (i, 0)),
                  pl.BlockSpec((1, 1, _BT), lambda i: (i, 0, 0)),
                  _const_spec(_D)],
        out_specs=[_const_spec(_D), _const_spec(1)],
        out_shape=[jax.ShapeDtypeStruct((_C, _D), f32),
                   jax.ShapeDtypeStruct((_C, 1), f32)],
    )(x, um_tc, pn)

    return pl.pallas_call(
        _k3_body,
        grid=(_NB + 1,),
        in_specs=[pl.BlockSpec((_B, _D), lambda i: (jnp.maximum(i - 1, 0), 0)),
                  pl.BlockSpec((_NSC, _D), lambda i: (_NTC // _NSC, 0)),
                  pl.BlockSpec((1, 1, _NSC), lambda i: (0, 0, 0)),
                  _const_spec(_NSC),
                  pl.BlockSpec((_NG, _C, 16), lambda i: (0, 0, 0)),
                  pl.BlockSpec((_NG, 16), lambda i: (0, 0)),
                  _const_spec(_D), _const_spec(1), _const_spec(_D), _const_spec(1)],
        out_specs=pl.BlockSpec((_B, _C), lambda i: (jnp.maximum(i - 1, 0), 0)),
        out_shape=jax.ShapeDtypeStruct((_N, _C), f32),
        scratch_shapes=[pltpu.VMEM((_C, _D), f32)],
    )(x, x, um_sc, probt_sc, skeys3, t3, tot, cnt, supx, colup)


# speedup vs baseline: 1.0699x; 1.0699x over previous
import functools

import jax
import jax.numpy as jnp
from jax.experimental import pallas as pl
from jax.experimental.pallas import tpu as pltpu
from jax.experimental.pallas import tpu_sc as plsc

_N = 10000
_D = 128
_C = 128
_N_LABELED = 1280
_K = int((_N - _N_LABELED) * 0.005)
_EPS = 1e-12
_INT_MIN = -(2 ** 31)
_B = 2000
_NB = _N // _B
_NSC = 400
_NTC = _N - _NSC
_BT = 1920
_NBT = _NTC // _BT
_NG = _NSC // 16
_NW = 32


def _rownorm(v):
    return v * jax.lax.rsqrt(jnp.maximum(jnp.sum(v * v, axis=1, keepdims=True), _EPS))


def _dot(a, b, ca, cb):
    return jax.lax.dot_general(
        a, b, (((ca,), (cb,)), ((), ())),
        preferred_element_type=jnp.float32,
        precision=jax.lax.Precision.HIGHEST,
    )


def _keys(probt):
    int_min = jnp.int32(_INT_MIN)
    i32 = jax.lax.bitcast_convert_type(probt, jnp.int32)
    return jnp.where(i32 >= 0, i32, jnp.bitwise_or(jnp.bitwise_not(i32), int_min))


def _kth_mask_t(probt):
    int_min = jnp.int32(_INT_MIN)
    s = _keys(probt)
    t = jnp.full((1, probt.shape[1]), _INT_MIN, jnp.int32)
    for b in range(31, -1, -1):
        inc = int_min if b == 31 else jnp.int32(1 << b)
        cand = t + inc
        cnt = jnp.sum((s >= cand).astype(jnp.int32), axis=0, keepdims=True)
        t = jnp.where(cnt >= _K, cand, t)
    return jnp.where(s >= t, probt, 0.0)


def _k1_body(x_ref, lab_ref, tot_ref, cnt_ref, pn_ref, probt_ref, skeys_ref):
    i = pl.program_id(0)

    @pl.when(i == 0)
    def _():
        tot_ref[...] = jnp.zeros_like(tot_ref)
        cnt_ref[...] = jnp.zeros_like(cnt_ref)

    x = x_ref[...]
    tot_ref[...] += _dot(lab_ref[...], x, 0, 0)
    cnt_ref[...] += _dot(lab_ref[...], jnp.ones((_B, 1), jnp.float32), 0, 0)

    @pl.when(i == _NB - 1)
    def _():
        protos = tot_ref[...] * (1.0 / cnt_ref[...])
        pn = _rownorm(protos)
        pn_ref[...] = pn
        xn_tail = _rownorm(x[_B - _NSC:])
        probt = _dot(pn, xn_tail, 1, 1)
        probt_ref[...] = probt
        skeys_ref[...] = _keys(probt)


def _topk_tc_body(x_ref, um_ref, pn_ref, supx_ref, colup_ref):
    i = pl.program_id(0)

    @pl.when(i == 0)
    def _():
        supx_ref[...] = jnp.zeros_like(supx_ref)
        colup_ref[...] = jnp.zeros_like(colup_ref)

    x = x_ref[...]
    xn = _rownorm(x)
    probt = _dot(pn_ref[...], xn, 1, 1)
    upt = _kth_mask_t(probt) * um_ref[0]
    supx_ref[...] += _dot(upt, x, 1, 0)
    colup_ref[...] += _dot(upt, jnp.ones((_BT, 1), jnp.float32), 1, 0)


def _k3_body(x_ref, xsc_ref, um_sc_ref, probt_ref, skeys_ref, t3_ref,
             tot_ref, cnt_ref, supx_ref, colup_ref, out_ref, pn2_ref):
    i = pl.program_id(0)

    @pl.when(i == 0)
    def _():
        probt = probt_ref[...]
        skeys = skeys_ref[...]
        pieces = []
        for g in range(_NG):
            t_bc = jnp.broadcast_to(t3_ref[g:g + 1, :], (_C, 16))
            pieces.append(jnp.where(skeys[:, 16 * g:16 * (g + 1)] >= t_bc,
                                    probt[:, 16 * g:16 * (g + 1)], 0.0))
        upt = jnp.concatenate(pieces, axis=1) * um_sc_ref[0]
        supx = supx_ref[...] + _dot(upt, xsc_ref[...], 1, 0)
        colup = colup_ref[...] + _dot(upt, jnp.ones((_NSC, 1), jnp.float32), 1, 0)
        countc = cnt_ref[...]
        protos = tot_ref[...] * (1.0 / countc)
        denom = colup + countc
        protos2 = protos + (supx - protos * colup) / denom
        pn2_ref[...] = _rownorm(protos2)

    out_ref[...] = _dot(_rownorm(x_ref[...]), pn2_ref[...], 1, 1)


def _sc_call(skeys):
    mesh = plsc.VectorSubcoreMesh(core_axis_name="c", subcore_axis_name="s")

    @functools.partial(
        pl.kernel,
        out_type=jax.ShapeDtypeStruct((_NG, 16), jnp.int32),
        mesh=mesh,
        scratch_types=[
            pltpu.VMEM((_C, _NSC), jnp.int32),
            pltpu.VMEM((16,), jnp.int32),
            pltpu.VMEM((16,), jnp.int32),
        ],
    )
    def sc_thresh(sk_hbm, t_hbm, buf, tbuf, cnt_ref):
        wid = jax.lax.axis_index("s") * 2 + jax.lax.axis_index("c")
        one = jnp.full((16,), 1, jnp.int32)
        zero = jnp.full((16,), 0, jnp.int32)
        pltpu.sync_copy(sk_hbm, buf)

        @pl.loop(0, (_NG + _NW - 1) // _NW)
        def _(j):
            g = wid + j * _NW

            @pl.when(g < _NG)
            def _():
                col = g * 16
                t = jnp.full((16,), _INT_MIN, jnp.int32)
                for b in range(31, -1, -1):
                    inc = jnp.int32(_INT_MIN) if b == 31 else jnp.int32(1 << b)
                    cand = t + inc
                    cnt_ref[...] = zero

                    @pl.loop(0, _C // 8)
                    def _(cc):
                        acc = zero
                        for u in range(8):
                            acc = acc + jnp.where(
                                buf[cc * 8 + u, pl.ds(col, 16)] >= cand, one, zero)
                        cnt_ref[...] += acc

                    t = jnp.where(cnt_ref[...] >= _K, cand, t)
                tbuf[...] = t
                pltpu.sync_copy(tbuf, t_hbm.at[g])

    return sc_thresh(skeys)


def _const_spec(r):
    return pl.BlockSpec((_C, r), lambda i: (0, 0))


def kernel(inputs, labels, labels_mask, unlabels_mask):
    del labels_mask
    f32 = jnp.float32
    um_tc = unlabels_mask[:_NTC].astype(f32).reshape(_NBT, 1, _BT)
    um_sc = unlabels_mask[_NTC:].astype(f32).reshape(1, 1, _NSC)
    x = inputs
    lab = labels.astype(f32)

    tot, cnt, pn, probt_sc, skeys_sc = pl.pallas_call(
        _k1_body,
        grid=(_NB,),
        in_specs=[pl.BlockSpec((_B, _D), lambda i: (i, 0)),
                  pl.BlockSpec((_B, _D), lambda i: (i, 0))],
        out_specs=[_const_spec(_D), _const_spec(1), _const_spec(_D),
                   _const_spec(_NSC), _const_spec(_NSC)],
        out_shape=[jax.ShapeDtypeStruct((_C, _D), f32),
                   jax.ShapeDtypeStruct((_C, 1), f32),
                   jax.ShapeDtypeStruct((_C, _D), f32),
                   jax.ShapeDtypeStruct((_C, _NSC), f32),
                   jax.ShapeDtypeStruct((_C, _NSC), jnp.int32)],
    )(x, lab)

    t3 = _sc_call(skeys_sc)

    supx, colup = pl.pallas_call(
        _topk_tc_body,
        grid=(_NBT,),
        in_specs=[pl.BlockSpec((_BT, _D), lambda i: (i, 0)),
                  pl.BlockSpec((1, 1, _BT), lambda i: (i, 0, 0)),
                  _const_spec(_D)],
        out_specs=[_const_spec(_D), _const_spec(1)],
        out_shape=[jax.ShapeDtypeStruct((_C, _D), f32),
                   jax.ShapeDtypeStruct((_C, 1), f32)],
    )(x, um_tc, pn)

    return pl.pallas_call(
        _k3_body,
        grid=(_NB + 1,),
        in_specs=[pl.BlockSpec((_B, _D), lambda i: (jnp.maximum(i - 1, 0), 0)),
                  pl.BlockSpec((_NSC, _D), lambda i: (_NTC // _NSC, 0)),
                  pl.BlockSpec((1, 1, _NSC), lambda i: (0, 0, 0)),
                  _const_spec(_NSC), _const_spec(_NSC),
                  pl.BlockSpec((_NG, 16), lambda i: (0, 0)),
                  _const_spec(_D), _const_spec(1), _const_spec(_D), _const_spec(1)],
        out_specs=pl.BlockSpec((_B, _C), lambda i: (jnp.maximum(i - 1, 0), 0)),
        out_shape=jax.ShapeDtypeStruct((_N, _C), f32),
        scratch_shapes=[pltpu.VMEM((_C, _D), f32)],
    )(x, x, um_sc, probt_sc, skeys_sc, t3, tot, cnt, supx, colup)


# speedup vs baseline: 1.2049x; 1.1262x over previous
import functools

import jax
import jax.numpy as jnp
from jax.experimental import pallas as pl
from jax.experimental.pallas import tpu as pltpu
from jax.experimental.pallas import tpu_sc as plsc

_N = 10000
_D = 128
_C = 128
_N_LABELED = 1280
_K = int((_N - _N_LABELED) * 0.005)
_EPS = 1e-12
_INT_MIN = -(2 ** 31)
_B = 2000
_NB = _N // _B
_NSC = 400
_NTC = _N - _NSC
_BT = 1920
_NBT = _NTC // _BT
_NG = _NSC // 16
_NW = 32


def _rownorm(v):
    return v * jax.lax.rsqrt(jnp.maximum(jnp.sum(v * v, axis=1, keepdims=True), _EPS))


def _dot1(a, b, ca, cb):
    return jax.lax.dot_general(
        a, b, (((ca,), (cb,)), ((), ())),
        preferred_element_type=jnp.float32,
    )


def _dot(a, b, ca, cb):
    bf16 = jnp.bfloat16
    ah = a.astype(bf16)
    al = (a - ah.astype(jnp.float32)).astype(bf16)
    bh = b.astype(bf16)
    bl = (b - bh.astype(jnp.float32)).astype(bf16)
    return (_dot1(ah, bh, ca, cb) + _dot1(ah, bl, ca, cb)
            + _dot1(al, bh, ca, cb))


def _keys(probt):
    int_min = jnp.int32(_INT_MIN)
    i32 = jax.lax.bitcast_convert_type(probt, jnp.int32)
    return jnp.where(i32 >= 0, i32, jnp.bitwise_or(jnp.bitwise_not(i32), int_min))


def _kth_mask_t(probt):
    int_min = jnp.int32(_INT_MIN)
    s = _keys(probt)
    t = jnp.full((1, probt.shape[1]), _INT_MIN, jnp.int32)
    for b in range(31, -1, -1):
        inc = int_min if b == 31 else jnp.int32(1 << b)
        cand = t + inc
        cnt = jnp.sum((s >= cand).astype(jnp.int32), axis=0, keepdims=True)
        t = jnp.where(cnt >= _K, cand, t)
    return jnp.where(s >= t, probt, 0.0)


def _k1_body(x_ref, lab_ref, tot_ref, cnt_ref, pn_ref, probt_ref, skeys_ref):
    i = pl.program_id(0)

    @pl.when(i == 0)
    def _():
        tot_ref[...] = jnp.zeros_like(tot_ref)
        cnt_ref[...] = jnp.zeros_like(cnt_ref)

    x = x_ref[...]
    tot_ref[...] += _dot(lab_ref[...], x, 0, 0)
    cnt_ref[...] += _dot(lab_ref[...], jnp.ones((_B, 1), jnp.float32), 0, 0)

    @pl.when(i == _NB - 1)
    def _():
        protos = tot_ref[...] * (1.0 / cnt_ref[...])
        pn = _rownorm(protos)
        pn_ref[...] = pn
        xn_tail = _rownorm(x[_B - _NSC:])
        probt = _dot(pn, xn_tail, 1, 1)
        probt_ref[...] = probt
        skeys_ref[...] = _keys(probt)


def _topk_tc_body(x_ref, um_ref, pn_ref, supx_ref, colup_ref):
    i = pl.program_id(0)

    @pl.when(i == 0)
    def _():
        supx_ref[...] = jnp.zeros_like(supx_ref)
        colup_ref[...] = jnp.zeros_like(colup_ref)

    x = x_ref[...]
    xn = _rownorm(x)
    probt = _dot(pn_ref[...], xn, 1, 1)
    upt = _kth_mask_t(probt) * um_ref[0]
    supx_ref[...] += _dot(upt, x, 1, 0)
    colup_ref[...] += _dot(upt, jnp.ones((_BT, 1), jnp.float32), 1, 0)


def _k3_body(x_ref, xsc_ref, um_sc_ref, probt_ref, skeys_ref, t3_ref,
             tot_ref, cnt_ref, supx_ref, colup_ref, out_ref, pn2_ref):
    i = pl.program_id(0)

    @pl.when(i == 0)
    def _():
        probt = probt_ref[...]
        skeys = skeys_ref[...]
        pieces = []
        for g in range(_NG):
            t_bc = jnp.broadcast_to(t3_ref[g:g + 1, :], (_C, 16))
            pieces.append(jnp.where(skeys[:, 16 * g:16 * (g + 1)] >= t_bc,
                                    probt[:, 16 * g:16 * (g + 1)], 0.0))
        upt = jnp.concatenate(pieces, axis=1) * um_sc_ref[0]
        supx = supx_ref[...] + _dot(upt, xsc_ref[...], 1, 0)
        colup = colup_ref[...] + _dot(upt, jnp.ones((_NSC, 1), jnp.float32), 1, 0)
        countc = cnt_ref[...]
        protos = tot_ref[...] * (1.0 / countc)
        denom = colup + countc
        protos2 = protos + (supx - protos * colup) / denom
        pn2_ref[...] = _rownorm(protos2)

    out_ref[...] = _dot(_rownorm(x_ref[...]), pn2_ref[...], 1, 1)


def _sc_call(skeys):
    mesh = plsc.VectorSubcoreMesh(core_axis_name="c", subcore_axis_name="s")

    @functools.partial(
        pl.kernel,
        out_type=jax.ShapeDtypeStruct((_NG, 16), jnp.int32),
        mesh=mesh,
        scratch_types=[
            pltpu.VMEM((_C, _NSC), jnp.int32),
            pltpu.VMEM((16,), jnp.int32),
            pltpu.VMEM((16,), jnp.int32),
        ],
    )
    def sc_thresh(sk_hbm, t_hbm, buf, tbuf, cnt_ref):
        wid = jax.lax.axis_index("s") * 2 + jax.lax.axis_index("c")
        one = jnp.full((16,), 1, jnp.int32)
        zero = jnp.full((16,), 0, jnp.int32)
        pltpu.sync_copy(sk_hbm, buf)

        @pl.loop(0, (_NG + _NW - 1) // _NW)
        def _(j):
            g = wid + j * _NW

            @pl.when(g < _NG)
            def _():
                col = g * 16
                t = jnp.full((16,), _INT_MIN, jnp.int32)
                for b in range(31, -1, -1):
                    inc = jnp.int32(_INT_MIN) if b == 31 else jnp.int32(1 << b)
                    cand = t + inc
                    cnt_ref[...] = zero

                    @pl.loop(0, _C // 8)
                    def _(cc):
                        acc = zero
                        for u in range(8):
                            acc = acc + jnp.where(
                                buf[cc * 8 + u, pl.ds(col, 16)] >= cand, one, zero)
                        cnt_ref[...] += acc

                    t = jnp.where(cnt_ref[...] >= _K, cand, t)
                tbuf[...] = t
                pltpu.sync_copy(tbuf, t_hbm.at[g])

    return sc_thresh(skeys)


def _const_spec(r):
    return pl.BlockSpec((_C, r), lambda i: (0, 0))


def kernel(inputs, labels, labels_mask, unlabels_mask):
    del labels_mask
    f32 = jnp.float32
    um_tc = unlabels_mask[:_NTC].astype(f32).reshape(_NBT, 1, _BT)
    um_sc = unlabels_mask[_NTC:].astype(f32).reshape(1, 1, _NSC)
    x = inputs
    lab = labels.astype(f32)

    tot, cnt, pn, probt_sc, skeys_sc = pl.pallas_call(
        _k1_body,
        grid=(_NB,),
        in_specs=[pl.BlockSpec((_B, _D), lambda i: (i, 0)),
                  pl.BlockSpec((_B, _D), lambda i: (i, 0))],
        out_specs=[_const_spec(_D), _const_spec(1), _const_spec(_D),
                   _const_spec(_NSC), _const_spec(_NSC)],
        out_shape=[jax.ShapeDtypeStruct((_C, _D), f32),
                   jax.ShapeDtypeStruct((_C, 1), f32),
                   jax.ShapeDtypeStruct((_C, _D), f32),
                   jax.ShapeDtypeStruct((_C, _NSC), f32),
                   jax.ShapeDtypeStruct((_C, _NSC), jnp.int32)],
    )(x, lab)

    t3 = _sc_call(skeys_sc)

    supx, colup = pl.pallas_call(
        _topk_tc_body,
        grid=(_NBT,),
        in_specs=[pl.BlockSpec((_BT, _D), lambda i: (i, 0)),
                  pl.BlockSpec((1, 1, _BT), lambda i: (i, 0, 0)),
                  _const_spec(_D)],
        out_specs=[_const_spec(_D), _const_spec(1)],
        out_shape=[jax.ShapeDtypeStruct((_C, _D), f32),
                   jax.ShapeDtypeStruct((_C, 1), f32)],
    )(x, um_tc, pn)

    return pl.pallas_call(
        _k3_body,
        grid=(_NB + 1,),
        in_specs=[pl.BlockSpec((_B, _D), lambda i: (jnp.maximum(i - 1, 0), 0)),
                  pl.BlockSpec((_NSC, _D), lambda i: (_NTC // _NSC, 0)),
                  pl.BlockSpec((1, 1, _NSC), lambda i: (0, 0, 0)),
                  _const_spec(_NSC), _const_spec(_NSC),
                  pl.BlockSpec((_NG, 16), lambda i: (0, 0)),
                  _const_spec(_D), _const_spec(1), _const_spec(_D), _const_spec(1)],
        out_specs=pl.BlockSpec((_B, _C), lambda i: (jnp.maximum(i - 1, 0), 0)),
        out_shape=jax.ShapeDtypeStruct((_N, _C), f32),
        scratch_shapes=[pltpu.VMEM((_C, _D), f32)],
    )(x, x, um_sc, probt_sc, skeys_sc, t3, tot, cnt, supx, colup)


# speedup vs baseline: 1.2890x; 1.0698x over previous
import functools

import jax
import jax.numpy as jnp
from jax.experimental import pallas as pl
from jax.experimental.pallas import tpu as pltpu
from jax.experimental.pallas import tpu_sc as plsc

_N = 10000
_D = 128
_C = 128
_N_LABELED = 1280
_K = int((_N - _N_LABELED) * 0.005)
_EPS = 1e-12
_INT_MIN = -(2 ** 31)
_B = 2000
_NB = _N // _B
_NSC = 400
_NTC = _N - _NSC
_BT = 1920
_NBT = _NTC // _BT
_NG = _NSC // 16
_NW = 32


def _rownorm(v):
    return v * jax.lax.rsqrt(jnp.maximum(jnp.sum(v * v, axis=1, keepdims=True), _EPS))


def _dot1(a, b, ca, cb):
    return jax.lax.dot_general(
        a, b, (((ca,), (cb,)), ((), ())),
        preferred_element_type=jnp.float32,
    )


def _dot(a, b, ca, cb):
    bf16 = jnp.bfloat16
    ah = a.astype(bf16)
    al = (a - ah.astype(jnp.float32)).astype(bf16)
    bh = b.astype(bf16)
    bl = (b - bh.astype(jnp.float32)).astype(bf16)
    return (_dot1(ah, bh, ca, cb) + _dot1(ah, bl, ca, cb)
            + _dot1(al, bh, ca, cb))


def _dot_xl(a, b, ca, cb):
    bf16 = jnp.bfloat16
    ah = a.astype(bf16)
    bh = b.astype(bf16)
    bl = (b - bh.astype(jnp.float32)).astype(bf16)
    return _dot1(ah, bh, ca, cb) + _dot1(ah, bl, ca, cb)


def _dot_xr(a, b, ca, cb):
    bf16 = jnp.bfloat16
    ah = a.astype(bf16)
    al = (a - ah.astype(jnp.float32)).astype(bf16)
    bh = b.astype(bf16)
    return _dot1(ah, bh, ca, cb) + _dot1(al, bh, ca, cb)


def _keys(probt):
    int_min = jnp.int32(_INT_MIN)
    i32 = jax.lax.bitcast_convert_type(probt, jnp.int32)
    return jnp.where(i32 >= 0, i32, jnp.bitwise_or(jnp.bitwise_not(i32), int_min))


def _kth_mask_t(probt):
    int_min = jnp.int32(_INT_MIN)
    s = _keys(probt)
    t = jnp.full((1, probt.shape[1]), _INT_MIN, jnp.int32)
    for b in range(31, -1, -1):
        inc = int_min if b == 31 else jnp.int32(1 << b)
        cand = t + inc
        cnt = jnp.sum((s >= cand).astype(jnp.int32), axis=0, keepdims=True)
        t = jnp.where(cnt >= _K, cand, t)
    return jnp.where(s >= t, probt, 0.0)


def _k1_body(x_ref, lab_ref, tot_ref, cnt_ref, pn_ref, probt_ref, skeys_ref):
    i = pl.program_id(0)

    @pl.when(i == 0)
    def _():
        tot_ref[...] = jnp.zeros_like(tot_ref)
        cnt_ref[...] = jnp.zeros_like(cnt_ref)

    x = x_ref[...]
    tot_ref[...] += _dot_xl(lab_ref[...], x, 0, 0)
    cnt_ref[...] += _dot1(lab_ref[...].astype(jnp.bfloat16),
                          jnp.ones((_B, 1), jnp.bfloat16), 0, 0)

    @pl.when(i == _NB - 1)
    def _():
        protos = tot_ref[...] * (1.0 / cnt_ref[...])
        pn = _rownorm(protos)
        pn_ref[...] = pn
        xn_tail = _rownorm(x[_B - _NSC:])
        probt = _dot(pn, xn_tail, 1, 1)
        probt_ref[...] = probt
        skeys_ref[...] = _keys(probt)


def _topk_tc_body(x_ref, um_ref, pn_ref, supx_ref, colup_ref):
    i = pl.program_id(0)

    @pl.when(i == 0)
    def _():
        supx_ref[...] = jnp.zeros_like(supx_ref)
        colup_ref[...] = jnp.zeros_like(colup_ref)

    x = x_ref[...]
    xn = _rownorm(x)
    probt = _dot(pn_ref[...], xn, 1, 1)
    upt = _kth_mask_t(probt) * um_ref[0]
    supx_ref[...] += _dot(upt, x, 1, 0)
    colup_ref[...] += _dot_xr(upt, jnp.ones((_BT, 1), jnp.float32), 1, 0)


def _k3_body(x_ref, xsc_ref, um_sc_ref, probt_ref, skeys_ref, t3_ref,
             tot_ref, cnt_ref, supx_ref, colup_ref, out_ref, pn2_ref):
    i = pl.program_id(0)

    @pl.when(i == 0)
    def _():
        probt = probt_ref[...]
        skeys = skeys_ref[...]
        pieces = []
        for g in range(_NG):
            t_bc = jnp.broadcast_to(t3_ref[g:g + 1, :], (_C, 16))
            pieces.append(jnp.where(skeys[:, 16 * g:16 * (g + 1)] >= t_bc,
                                    probt[:, 16 * g:16 * (g + 1)], 0.0))
        upt = jnp.concatenate(pieces, axis=1) * um_sc_ref[0]
        supx = supx_ref[...] + _dot(upt, xsc_ref[...], 1, 0)
        colup = colup_ref[...] + _dot_xr(upt, jnp.ones((_NSC, 1), jnp.float32), 1, 0)
        countc = cnt_ref[...]
        protos = tot_ref[...] * (1.0 / countc)
        denom = colup + countc
        protos2 = protos + (supx - protos * colup) / denom
        pn2_ref[...] = _rownorm(protos2)

    out_ref[...] = _dot(_rownorm(x_ref[...]), pn2_ref[...], 1, 1)


def _sc_call(skeys):
    mesh = plsc.VectorSubcoreMesh(core_axis_name="c", subcore_axis_name="s")

    @functools.partial(
        pl.kernel,
        out_type=jax.ShapeDtypeStruct((_NG, 16), jnp.int32),
        mesh=mesh,
        scratch_types=[
            pltpu.VMEM((_C, _NSC), jnp.int32),
            pltpu.VMEM((16,), jnp.int32),
            pltpu.VMEM((16,), jnp.int32),
        ],
    )
    def sc_thresh(sk_hbm, t_hbm, buf, tbuf, cnt_ref):
        wid = jax.lax.axis_index("s") * 2 + jax.lax.axis_index("c")
        one = jnp.full((16,), 1, jnp.int32)
        zero = jnp.full((16,), 0, jnp.int32)
        pltpu.sync_copy(sk_hbm, buf)

        @pl.loop(0, (_NG + _NW - 1) // _NW)
        def _(j):
            g = wid + j * _NW

            @pl.when(g < _NG)
            def _():
                col = g * 16
                t = jnp.full((16,), _INT_MIN, jnp.int32)
                for b in range(31, -1, -1):
                    inc = jnp.int32(_INT_MIN) if b == 31 else jnp.int32(1 << b)
                    cand = t + inc
                    cnt_ref[...] = zero

                    @pl.loop(0, _C // 8)
                    def _(cc):
                        acc = zero
                        for u in range(8):
                            acc = acc + jnp.where(
                                buf[cc * 8 + u, pl.ds(col, 16)] >= cand, one, zero)
                        cnt_ref[...] += acc

                    t = jnp.where(cnt_ref[...] >= _K, cand, t)
                tbuf[...] = t
                pltpu.sync_copy(tbuf, t_hbm.at[g])

    return sc_thresh(skeys)


def _const_spec(r):
    return pl.BlockSpec((_C, r), lambda i: (0, 0))


def kernel(inputs, labels, labels_mask, unlabels_mask):
    del labels_mask
    f32 = jnp.float32
    um_tc = unlabels_mask[:_NTC].astype(f32).reshape(_NBT, 1, _BT)
    um_sc = unlabels_mask[_NTC:].astype(f32).reshape(1, 1, _NSC)
    x = inputs
    lab = labels.astype(f32)

    tot, cnt, pn, probt_sc, skeys_sc = pl.pallas_call(
        _k1_body,
        grid=(_NB,),
        in_specs=[pl.BlockSpec((_B, _D), lambda i: (i, 0)),
                  pl.BlockSpec((_B, _D), lambda i: (i, 0))],
        out_specs=[_const_spec(_D), _const_spec(1), _const_spec(_D),
                   _const_spec(_NSC), _const_spec(_NSC)],
        out_shape=[jax.ShapeDtypeStruct((_C, _D), f32),
                   jax.ShapeDtypeStruct((_C, 1), f32),
                   jax.ShapeDtypeStruct((_C, _D), f32),
                   jax.ShapeDtypeStruct((_C, _NSC), f32),
                   jax.ShapeDtypeStruct((_C, _NSC), jnp.int32)],
    )(x, lab)

    t3 = _sc_call(skeys_sc)

    supx, colup = pl.pallas_call(
        _topk_tc_body,
        grid=(_NBT,),
        in_specs=[pl.BlockSpec((_BT, _D), lambda i: (i, 0)),
                  pl.BlockSpec((1, 1, _BT), lambda i: (i, 0, 0)),
                  _const_spec(_D)],
        out_specs=[_const_spec(_D), _const_spec(1)],
        out_shape=[jax.ShapeDtypeStruct((_C, _D), f32),
                   jax.ShapeDtypeStruct((_C, 1), f32)],
    )(x, um_tc, pn)

    return pl.pallas_call(
        _k3_body,
        grid=(_NB,),
        in_specs=[pl.BlockSpec((_B, _D), lambda i: (i, 0)),
                  pl.BlockSpec((_NSC, _D), lambda i: (_NTC // _NSC, 0)),
                  pl.BlockSpec((1, 1, _NSC), lambda i: (0, 0, 0)),
                  _const_spec(_NSC), _const_spec(_NSC),
                  pl.BlockSpec((_NG, 16), lambda i: (0, 0)),
                  _const_spec(_D), _const_spec(1), _const_spec(_D), _const_spec(1)],
        out_specs=pl.BlockSpec((_B, _C), lambda i: (i, 0)),
        out_shape=jax.ShapeDtypeStruct((_N, _C), f32),
        scratch_shapes=[pltpu.VMEM((_C, _D), f32)],
    )(x, x, um_sc, probt_sc, skeys_sc, t3, tot, cnt, supx, colup)


# speedup vs baseline: 1.4381x; 1.1156x over previous
import functools

import jax
import jax.numpy as jnp
from jax.experimental import pallas as pl
from jax.experimental.pallas import tpu as pltpu
from jax.experimental.pallas import tpu_sc as plsc

_N = 10000
_D = 128
_C = 128
_N_LABELED = 1280
_K = int((_N - _N_LABELED) * 0.005)
_EPS = 1e-12
_INT_MIN = -(2 ** 31)
_B = 2000
_NB = _N // _B
_NSC = 400
_NTC = _N - _NSC
_BT = 1920
_NBT = _NTC // _BT
_NG = _NSC // 16
_NW = 32


def _rownorm(v):
    return v * jax.lax.rsqrt(jnp.maximum(jnp.sum(v * v, axis=1, keepdims=True), _EPS))


def _dot1(a, b, ca, cb):
    return jax.lax.dot_general(
        a, b, (((ca,), (cb,)), ((), ())),
        preferred_element_type=jnp.float32,
    )


def _dot(a, b, ca, cb):
    bf16 = jnp.bfloat16
    ah = a.astype(bf16)
    al = (a - ah.astype(jnp.float32)).astype(bf16)
    bh = b.astype(bf16)
    bl = (b - bh.astype(jnp.float32)).astype(bf16)
    return (_dot1(ah, bh, ca, cb) + _dot1(ah, bl, ca, cb)
            + _dot1(al, bh, ca, cb))


def _dot_xl(a, b, ca, cb):
    bf16 = jnp.bfloat16
    ah = a.astype(bf16)
    bh = b.astype(bf16)
    bl = (b - bh.astype(jnp.float32)).astype(bf16)
    return _dot1(ah, bh, ca, cb) + _dot1(ah, bl, ca, cb)


def _dot_xr(a, b, ca, cb):
    bf16 = jnp.bfloat16
    ah = a.astype(bf16)
    al = (a - ah.astype(jnp.float32)).astype(bf16)
    bh = b.astype(bf16)
    return _dot1(ah, bh, ca, cb) + _dot1(al, bh, ca, cb)


def _keys(probt):
    int_min = jnp.int32(_INT_MIN)
    i32 = jax.lax.bitcast_convert_type(probt, jnp.int32)
    return jnp.where(i32 >= 0, i32, jnp.bitwise_or(jnp.bitwise_not(i32), int_min))


def _kth_mask_t(probt):
    cols = probt.shape[1]
    s = _keys(probt)
    i16 = jnp.int16

    def _colsum16(mask):
        v = mask
        while v.shape[0] > 16:
            h = v.shape[0] // 2
            v = v[:h] + v[h:]
        v32 = jnp.sum(v.astype(jnp.int32), axis=0, keepdims=True)
        return v32.astype(i16)
    hi = jax.lax.shift_right_arithmetic(s, 16).astype(i16)
    t_hi = jnp.full((1, cols), -32768, i16)
    for b in range(15, -1, -1):
        inc = jnp.int16(-32768) if b == 15 else jnp.int16(1 << b)
        cand = t_hi + inc
        cnt = _colsum16((hi >= cand).astype(i16))
        ge = (cnt >= jnp.int16(_K)).astype(i16)
        t_hi = t_hi + inc * ge
    lo_u = jnp.bitwise_and(s, jnp.int32(0xFFFF))
    lo = jnp.bitwise_xor(lo_u, jnp.int32(0x8000)).astype(i16)
    adj = jnp.where(hi > t_hi, jnp.int16(32767),
                    jnp.where(hi < t_hi, jnp.int16(-32768), lo))
    t_lo = jnp.full((1, cols), -32768, i16)
    for b in range(15, -1, -1):
        inc = jnp.int16(-32768) if b == 15 else jnp.int16(1 << b)
        cand = t_lo + inc
        cnt = _colsum16((adj >= cand).astype(i16))
        ge = (cnt >= jnp.int16(_K)).astype(i16)
        t_lo = t_lo + inc * ge
    t32 = jnp.bitwise_or(
        jax.lax.shift_left(t_hi.astype(jnp.int32), 16),
        jnp.bitwise_xor(t_lo.astype(jnp.int32) & jnp.int32(0xFFFF),
                        jnp.int32(0x8000)))
    return jnp.where(s >= t32, probt, 0.0)


def _k1_body(x_ref, lab_ref, tot_ref, cnt_ref, pn_ref, probt_ref, skeys_ref):
    i = pl.program_id(0)

    @pl.when(i == 0)
    def _():
        tot_ref[...] = jnp.zeros_like(tot_ref)
        cnt_ref[...] = jnp.zeros_like(cnt_ref)

    x = x_ref[...]
    tot_ref[...] += _dot_xl(lab_ref[...], x, 0, 0)
    cnt_ref[...] += _dot1(lab_ref[...].astype(jnp.bfloat16),
                          jnp.ones((_B, 1), jnp.bfloat16), 0, 0)

    @pl.when(i == _NB - 1)
    def _():
        protos = tot_ref[...] * (1.0 / cnt_ref[...])
        pn = _rownorm(protos)
        pn_ref[...] = pn
        xn_tail = _rownorm(x[_B - _NSC:])
        probt = _dot(pn, xn_tail, 1, 1)
        probt_ref[...] = probt
        skeys_ref[...] = _keys(probt)


def _topk_tc_body(x_ref, um_ref, pn_ref, supx_ref, colup_ref):
    i = pl.program_id(0)

    @pl.when(i == 0)
    def _():
        supx_ref[...] = jnp.zeros_like(supx_ref)
        colup_ref[...] = jnp.zeros_like(colup_ref)

    x = x_ref[...]
    xn = _rownorm(x)
    probt = _dot(pn_ref[...], xn, 1, 1)
    upt = _kth_mask_t(probt) * um_ref[0]
    supx_ref[...] += _dot(upt, x, 1, 0)
    colup_ref[...] += _dot_xr(upt, jnp.ones((_BT, 1), jnp.float32), 1, 0)


def _k3_body(x_ref, xsc_ref, um_sc_ref, probt_ref, skeys_ref, t3_ref,
             tot_ref, cnt_ref, supx_ref, colup_ref, out_ref, pn2_ref):
    i = pl.program_id(0)

    @pl.when(i == 0)
    def _():
        probt = probt_ref[...]
        skeys = skeys_ref[...]
        pieces = []
        for g in range(_NG):
            t_bc = jnp.broadcast_to(t3_ref[g:g + 1, :], (_C, 16))
            pieces.append(jnp.where(skeys[:, 16 * g:16 * (g + 1)] >= t_bc,
                                    probt[:, 16 * g:16 * (g + 1)], 0.0))
        upt = jnp.concatenate(pieces, axis=1) * um_sc_ref[0]
        supx = supx_ref[...] + _dot(upt, xsc_ref[...], 1, 0)
        colup = colup_ref[...] + _dot_xr(upt, jnp.ones((_NSC, 1), jnp.float32), 1, 0)
        countc = cnt_ref[...]
        protos = tot_ref[...] * (1.0 / countc)
        denom = colup + countc
        protos2 = protos + (supx - protos * colup) / denom
        pn2_ref[...] = _rownorm(protos2)

    out_ref[...] = _dot(_rownorm(x_ref[...]), pn2_ref[...], 1, 1)


def _sc_call(skeys):
    mesh = plsc.VectorSubcoreMesh(core_axis_name="c", subcore_axis_name="s")

    @functools.partial(
        pl.kernel,
        out_type=jax.ShapeDtypeStruct((_NG, 16), jnp.int32),
        mesh=mesh,
        scratch_types=[
            pltpu.VMEM((_C, _NSC), jnp.int32),
            pltpu.VMEM((16,), jnp.int32),
            pltpu.VMEM((16,), jnp.int32),
        ],
    )
    def sc_thresh(sk_hbm, t_hbm, buf, tbuf, cnt_ref):
        wid = jax.lax.axis_index("s") * 2 + jax.lax.axis_index("c")
        one = jnp.full((16,), 1, jnp.int32)
        zero = jnp.full((16,), 0, jnp.int32)
        pltpu.sync_copy(sk_hbm, buf)

        @pl.loop(0, (_NG + _NW - 1) // _NW)
        def _(j):
            g = wid + j * _NW

            @pl.when(g < _NG)
            def _():
                col = g * 16
                t = jnp.full((16,), _INT_MIN, jnp.int32)
                for b in range(31, -1, -1):
                    inc = jnp.int32(_INT_MIN) if b == 31 else jnp.int32(1 << b)
                    cand = t + inc
                    cnt_ref[...] = zero

                    @pl.loop(0, _C // 8)
                    def _(cc):
                        acc = zero
                        for u in range(8):
                            acc = acc + jnp.where(
                                buf[cc * 8 + u, pl.ds(col, 16)] >= cand, one, zero)
                        cnt_ref[...] += acc

                    t = jnp.where(cnt_ref[...] >= _K, cand, t)
                tbuf[...] = t
                pltpu.sync_copy(tbuf, t_hbm.at[g])

    return sc_thresh(skeys)


def _const_spec(r):
    return pl.BlockSpec((_C, r), lambda i: (0, 0))


def kernel(inputs, labels, labels_mask, unlabels_mask):
    del labels_mask
    f32 = jnp.float32
    um_tc = unlabels_mask[:_NTC].astype(f32).reshape(_NBT, 1, _BT)
    um_sc = unlabels_mask[_NTC:].astype(f32).reshape(1, 1, _NSC)
    x = inputs
    lab = labels.astype(f32)

    tot, cnt, pn, probt_sc, skeys_sc = pl.pallas_call(
        _k1_body,
        grid=(_NB,),
        in_specs=[pl.BlockSpec((_B, _D), lambda i: (i, 0)),
                  pl.BlockSpec((_B, _D), lambda i: (i, 0))],
        out_specs=[_const_spec(_D), _const_spec(1), _const_spec(_D),
                   _const_spec(_NSC), _const_spec(_NSC)],
        out_shape=[jax.ShapeDtypeStruct((_C, _D), f32),
                   jax.ShapeDtypeStruct((_C, 1), f32),
                   jax.ShapeDtypeStruct((_C, _D), f32),
                   jax.ShapeDtypeStruct((_C, _NSC), f32),
                   jax.ShapeDtypeStruct((_C, _NSC), jnp.int32)],
    )(x, lab)

    t3 = _sc_call(skeys_sc)

    supx, colup = pl.pallas_call(
        _topk_tc_body,
        grid=(_NBT,),
        in_specs=[pl.BlockSpec((_BT, _D), lambda i: (i, 0)),
                  pl.BlockSpec((1, 1, _BT), lambda i: (i, 0, 0)),
                  _const_spec(_D)],
        out_specs=[_const_spec(_D), _const_spec(1)],
        out_shape=[jax.ShapeDtypeStruct((_C, _D), f32),
                   jax.ShapeDtypeStruct((_C, 1), f32)],
    )(x, um_tc, pn)

    return pl.pallas_call(
        _k3_body,
        grid=(_NB,),
        in_specs=[pl.BlockSpec((_B, _D), lambda i: (i, 0)),
                  pl.BlockSpec((_NSC, _D), lambda i: (_NTC // _NSC, 0)),
                  pl.BlockSpec((1, 1, _NSC), lambda i: (0, 0, 0)),
                  _const_spec(_NSC), _const_spec(_NSC),
                  pl.BlockSpec((_NG, 16), lambda i: (0, 0)),
                  _const_spec(_D), _const_spec(1), _const_spec(_D), _const_spec(1)],
        out_specs=pl.BlockSpec((_B, _C), lambda i: (i, 0)),
        out_shape=jax.ShapeDtypeStruct((_N, _C), f32),
        scratch_shapes=[pltpu.VMEM((_C, _D), f32)],
    )(x, x, um_sc, probt_sc, skeys_sc, t3, tot, cnt, supx, colup)
